# Initial kernel scaffold; baseline (speedup 1.0000x reference)
import jax
import jax.numpy as jnp
from jax.experimental import pallas as pl


def kernel(x, edge_index, Wn1, Wr1, b1, g1, be1, Wn2, Wr2, b2, g2, be2, Wn3, Wr3, b3):
    raise NotImplementedError("write your pallas kernel here")


# speedup vs baseline: 2.9349x; 2.9349x over previous
import jax
import jax.numpy as jnp
from jax import lax
from jax.experimental import pallas as pl
from jax.experimental.pallas import tpu as pltpu
from jax.experimental.pallas import tpu_sc as plsc

N = 10000
E = 160000
D_IN = 256
H1 = 512
H2 = 256
D_OUT = 128
EPS_BN = 1e-5

NS = 16
NPAD = 10240
RPS = NPAD // NS
K = 128
EPAD = 163840
EPS_SC = EPAD // NS
NIT = EPS_SC // K
DEGW = 128

R_TC = 2000
G_TC = N // R_TC

_MESH = plsc.VectorSubcoreMesh(core_axis_name="c", subcore_axis_name="s")


NWIN = EPAD // K


def _make_sc_agg(W, esplit, with_deg=False):
    scratch = [
        pltpu.VMEM((K,), jnp.int32),
        pltpu.VMEM((K, W), jnp.float32),
        pltpu.VMEM_SHARED((NPAD, W), jnp.float32),
        pltpu.SemaphoreType.DMA,
    ]
    outs = [jax.ShapeDtypeStruct((2 * NPAD, W), jnp.float32)]
    if with_deg:
        scratch += [
            pltpu.VMEM((K, DEGW), jnp.float32),
            pltpu.VMEM_SHARED((NPAD, DEGW), jnp.float32),
        ]
        outs.append(jax.ShapeDtypeStruct((2 * NPAD, DEGW), jnp.float32))

    def body_fn(tbl, s_hbm, d_hbm, zW, *rest):
        if with_deg:
            (zD, ones_hbm, out, outD, src2_v, rows_v, acc_sh, sem,
             ones_v, dacc_sh) = rest
        else:
            out, src2_v, rows_v, acc_sh, sem = rest
        c = lax.axis_index("c")
        s = lax.axis_index("s")
        off_r = s * RPS

        pltpu.async_copy(zW.at[pl.ds(off_r, RPS)],
                         acc_sh.at[pl.ds(off_r, RPS)], sem).wait()
        if with_deg:
            pltpu.async_copy(ones_hbm, ones_v, sem).wait()
            pltpu.async_copy(zD.at[pl.ds(off_r, RPS)],
                             dacc_sh.at[pl.ds(off_r, RPS)], sem).wait()
        plsc.subcore_barrier()

        base = c * N

        def win(si_vmem, di_vmem):
            if esplit:
                pltpu.sync_copy(tbl.at[si_vmem.at[0]], rows_v)
            else:
                @pl.loop(0, K, step=16)
                def _(j):
                    src2_v[pl.ds(j, 16)] = si_vmem[0, pl.ds(j, 16)] + base
                pltpu.sync_copy(tbl.at[src2_v], rows_v)
            pltpu.sync_copy(rows_v, acc_sh.at[di_vmem.at[0]], add=True)
            if with_deg:
                pltpu.sync_copy(ones_v, dacc_sh.at[di_vmem.at[0]], add=True)

        pltpu.emit_pipeline(
            win,
            grid=(NWIN,),
            in_specs=[
                pl.BlockSpec((1, K), lambda i: (0, i)),
                pl.BlockSpec((1, K), lambda i: (0, i)),
            ],
            out_specs=[],
            core_axis_name=("c", "s") if esplit else "s",
            dimension_semantics=(pltpu.PARALLEL,),
        )(s_hbm, d_hbm)
        plsc.subcore_barrier()

        pltpu.async_copy(acc_sh.at[pl.ds(off_r, RPS)],
                         out.at[pl.ds(c * NPAD + off_r, RPS)], sem).wait()
        if with_deg:
            pltpu.async_copy(dacc_sh.at[pl.ds(off_r, RPS)],
                             outD.at[pl.ds(c * NPAD + off_r, RPS)], sem).wait()

    return pl.kernel(body_fn,
                     out_type=tuple(outs) if with_deg else outs[0],
                     mesh=_MESH, scratch_types=scratch)


_sc_agg1 = _make_sc_agg(D_IN // 2, esplit=False)
_sc_deg = _make_sc_agg(DEGW, esplit=True)
_sc_agg2 = _make_sc_agg(H2 // 2, esplit=False)
_sc_agg3 = _make_sc_agg(D_OUT, esplit=True)


def _l1_body(x_ref, a_ref, deg_ref, WnT_ref, WrT_ref, b_ref,
             r_ref, s_ref, q_ref):
    i = pl.program_id(0)
    inv = 1.0 / jnp.maximum(deg_ref[0][:, 0:1] + deg_ref[1][:, 0:1], 1.0)
    m = jnp.concatenate([a_ref[0], a_ref[1]], axis=1) * inv
    z = jnp.dot(m, WnT_ref[...], preferred_element_type=jnp.float32)
    z += jnp.dot(x_ref[...], WrT_ref[...], preferred_element_type=jnp.float32)
    z += b_ref[...]
    r = jnp.maximum(z, 0.0)
    r_ref[...] = r

    @pl.when(i == 0)
    def _():
        s_ref[...] = jnp.zeros_like(s_ref)
        q_ref[...] = jnp.zeros_like(q_ref)
    s_ref[...] += jnp.sum(r, axis=0, keepdims=True)
    q_ref[...] += jnp.sum(r * r, axis=0, keepdims=True)


def _tc_l1(x, a, deg, WnT, WrT, b):
    H = WnT.shape[1]
    W = a.shape[2]
    return pl.pallas_call(
        _l1_body,
        grid=(G_TC,),
        in_specs=[
            pl.BlockSpec((R_TC, D_IN), lambda i: (i, 0)),
            pl.BlockSpec((2, R_TC, W), lambda i: (0, i, 0)),
            pl.BlockSpec((2, R_TC, DEGW), lambda i: (0, i, 0)),
            pl.BlockSpec((D_IN, H), lambda i: (0, 0)),
            pl.BlockSpec((D_IN, H), lambda i: (0, 0)),
            pl.BlockSpec((1, H), lambda i: (0, 0)),
        ],
        out_specs=[
            pl.BlockSpec((R_TC, H), lambda i: (i, 0)),
            pl.BlockSpec((1, H), lambda i: (0, 0)),
            pl.BlockSpec((1, H), lambda i: (0, 0)),
        ],
        out_shape=[
            jax.ShapeDtypeStruct((N, H), jnp.float32),
            jax.ShapeDtypeStruct((1, H), jnp.float32),
            jax.ShapeDtypeStruct((1, H), jnp.float32),
        ],
    )(x, a, deg, WnT, WrT, b)


def _mm_body_split(r_ref, s_ref, q_ref, g_ref, be_ref, WnT_ref, WrT_ref,
                   yn_ref, yr_ref):
    mu = s_ref[...] * (1.0 / N)
    var = q_ref[...] * (1.0 / N) - mu * mu
    scale = g_ref[...] * lax.rsqrt(var + EPS_BN)
    shift = be_ref[...] - mu * scale
    h = r_ref[...] * scale + shift
    yn = jnp.dot(h, WnT_ref[...], preferred_element_type=jnp.float32)
    yr = jnp.dot(h, WrT_ref[...], preferred_element_type=jnp.float32)
    half = yn_ref.shape[2]
    yn_ref[0] = yn[:, :half]
    yn_ref[1] = yn[:, half:]
    yr_ref[...] = yr


def _mm_body_whole(r_ref, s_ref, q_ref, g_ref, be_ref, WnT_ref, WrT_ref,
                   yn_ref, yr_ref):
    mu = s_ref[...] * (1.0 / N)
    var = q_ref[...] * (1.0 / N) - mu * mu
    scale = g_ref[...] * lax.rsqrt(var + EPS_BN)
    shift = be_ref[...] - mu * scale
    h = r_ref[...] * scale + shift
    yn_ref[...] = jnp.dot(h, WnT_ref[...], preferred_element_type=jnp.float32)
    yr_ref[...] = jnp.dot(h, WrT_ref[...], preferred_element_type=jnp.float32)


def _tc_mm(r, s, q, g, be, WnT, WrT, split):
    H = WnT.shape[0]
    H2o = WnT.shape[1]
    half = H2o // 2
    if split:
        out_specs = [
            pl.BlockSpec((2, R_TC, half), lambda i: (0, i, 0)),
            pl.BlockSpec((R_TC, H2o), lambda i: (i, 0)),
        ]
        out_shape = [
            jax.ShapeDtypeStruct((2, N, half), jnp.float32),
            jax.ShapeDtypeStruct((N, H2o), jnp.float32),
        ]
        body = _mm_body_split
    else:
        out_specs = [
            pl.BlockSpec((R_TC, H2o), lambda i: (i, 0)),
            pl.BlockSpec((R_TC, H2o), lambda i: (i, 0)),
        ]
        out_shape = [
            jax.ShapeDtypeStruct((N, H2o), jnp.float32),
            jax.ShapeDtypeStruct((N, H2o), jnp.float32),
        ]
        body = _mm_body_whole
    return pl.pallas_call(
        body,
        grid=(G_TC,),
        in_specs=[
            pl.BlockSpec((R_TC, H), lambda i: (i, 0)),
            pl.BlockSpec((1, H), lambda i: (0, 0)),
            pl.BlockSpec((1, H), lambda i: (0, 0)),
            pl.BlockSpec((1, H), lambda i: (0, 0)),
            pl.BlockSpec((1, H), lambda i: (0, 0)),
            pl.BlockSpec((H, H2o), lambda i: (0, 0)),
            pl.BlockSpec((H, H2o), lambda i: (0, 0)),
        ],
        out_specs=out_specs,
        out_shape=out_shape,
    )(r, s, q, g, be, WnT, WrT)


def _comb_body(a_ref, deg_ref, yr_ref, b_ref, r_ref, s_ref, q_ref):
    i = pl.program_id(0)
    inv = 1.0 / jnp.maximum(deg_ref[0][:, 0:1] + deg_ref[1][:, 0:1], 1.0)
    m = jnp.concatenate([a_ref[0], a_ref[1]], axis=1) * inv
    z = m + yr_ref[...] + b_ref[...]
    r = jnp.maximum(z, 0.0)
    r_ref[...] = r

    @pl.when(i == 0)
    def _():
        s_ref[...] = jnp.zeros_like(s_ref)
        q_ref[...] = jnp.zeros_like(q_ref)
    s_ref[...] += jnp.sum(r, axis=0, keepdims=True)
    q_ref[...] += jnp.sum(r * r, axis=0, keepdims=True)


def _tc_comb(a, deg, yr, b):
    W = a.shape[2]
    H = 2 * W
    return pl.pallas_call(
        _comb_body,
        grid=(G_TC,),
        in_specs=[
            pl.BlockSpec((2, R_TC, W), lambda i: (0, i, 0)),
            pl.BlockSpec((2, R_TC, DEGW), lambda i: (0, i, 0)),
            pl.BlockSpec((R_TC, H), lambda i: (i, 0)),
            pl.BlockSpec((1, H), lambda i: (0, 0)),
        ],
        out_specs=[
            pl.BlockSpec((R_TC, H), lambda i: (i, 0)),
            pl.BlockSpec((1, H), lambda i: (0, 0)),
            pl.BlockSpec((1, H), lambda i: (0, 0)),
        ],
        out_shape=[
            jax.ShapeDtypeStruct((N, H), jnp.float32),
            jax.ShapeDtypeStruct((1, H), jnp.float32),
            jax.ShapeDtypeStruct((1, H), jnp.float32),
        ],
    )(a, deg, yr, b)


def _fin_body(a_ref, deg_ref, yr_ref, b_ref, o_ref):
    inv = 1.0 / jnp.maximum(deg_ref[0][:, 0:1] + deg_ref[1][:, 0:1], 1.0)
    m = (a_ref[0] + a_ref[1]) * inv
    o_ref[...] = m + yr_ref[...] + b_ref[...]


def _tc_fin(a, deg, yr, b):
    H = a.shape[2]
    return pl.pallas_call(
        _fin_body,
        grid=(G_TC,),
        in_specs=[
            pl.BlockSpec((2, R_TC, H), lambda i: (0, i, 0)),
            pl.BlockSpec((2, R_TC, DEGW), lambda i: (0, i, 0)),
            pl.BlockSpec((R_TC, H), lambda i: (i, 0)),
            pl.BlockSpec((1, H), lambda i: (0, 0)),
        ],
        out_specs=pl.BlockSpec((R_TC, H), lambda i: (i, 0)),
        out_shape=jax.ShapeDtypeStruct((N, H), jnp.float32),
    )(a, deg, yr, b)


def kernel(x, edge_index, Wn1, Wr1, b1, g1, be1, Wn2, Wr2, b2, g2, be2,
           Wn3, Wr3, b3):
    src = edge_index[0].astype(jnp.int32)
    dst = edge_index[1].astype(jnp.int32)
    src_p = jnp.concatenate([src, jnp.zeros((EPAD - E,), jnp.int32)])
    dst_p = jnp.concatenate([dst, jnp.full((EPAD - E,), NPAD - 1, jnp.int32)])

    src2 = src_p.reshape(1, EPAD)
    dst2 = dst_p.reshape(1, EPAD)

    half_in = D_IN // 2
    xAB = jnp.stack([x[:, :half_in], x[:, half_in:]]).reshape(2 * N, half_in)
    zW1 = jnp.zeros((NPAD, half_in), jnp.float32)
    zD = jnp.zeros((NPAD, DEGW), jnp.float32)
    zW3 = jnp.zeros((NPAD, D_OUT), jnp.float32)

    a1 = _sc_agg1(xAB, src2, dst2, zW1).reshape(2, NPAD, half_in)
    dega = _sc_deg(jnp.ones((N, DEGW), jnp.float32), src2, dst2,
                   zD).reshape(2, NPAD, DEGW)
    r1, s1, q1 = _tc_l1(x, a1, dega, Wn1.T, Wr1.T, b1.reshape(1, -1))
    y2, y2r = _tc_mm(r1, s1, q1, g1.reshape(1, -1), be1.reshape(1, -1),
                     Wn2.T, Wr2.T, split=True)
    a2 = _sc_agg2(y2.reshape(2 * N, -1), src2, dst2,
                  zW1).reshape(2, NPAD, half_in)
    r2, s2, q2 = _tc_comb(a2, dega, y2r, b2.reshape(1, -1))
    y3n, y3r = _tc_mm(r2, s2, q2, g2.reshape(1, -1), be2.reshape(1, -1),
                      Wn3.T, Wr3.T, split=False)
    a3 = _sc_agg3(y3n, src2, dst2, zW3).reshape(2, NPAD, D_OUT)
    return _tc_fin(a3, dega, y3r, b3.reshape(1, -1))
